# Initial kernel scaffold; baseline (speedup 1.0000x reference)
import jax
import jax.numpy as jnp
from jax.experimental import pallas as pl


def kernel(x, edge_index, edge_attr, l1_W, l1_att_src, l1_att_dst, l1_We, l1_att_edge, l1_b, l2_W, l2_att_src, l2_att_dst, l2_We, l2_att_edge, l2_b, ln_gamma, ln_beta):
    raise NotImplementedError("write your pallas kernel here")


# speedup vs baseline: 23.5439x; 23.5439x over previous
import functools

import jax
import jax.numpy as jnp
from jax import lax
from jax.experimental import pallas as pl
from jax.experimental.pallas import tpu as pltpu
from jax.experimental.pallas import tpu_sc as plsc

_N = 10000
_E = 320000
_D = 128
_DE = 16

_NB = 10
_BN = _N // _NB
_BEA = (_E * _DE // _D) // _NB

_NT = 32
_EW = _E // _NT
_CH = 80
_NC = _EW // _CH
_RPT = _N // 16


def _lrelu(t):
    return jnp.where(t >= 0.0, t, 0.2 * t)


def _a1_body(x_ref, ea_ref, w_ref, att_ref, ve_ref,
             h_ref, asd_ref, ae_ref, s1_ref, mx2_ref, mx_s):
    i = pl.program_id(0)
    hb = x_ref[...] @ w_ref[...]
    h_ref[...] = hb
    asd = hb @ att_ref[...]
    asd_ref[...] = asd
    p = ea_ref[...] @ ve_ref[...]
    ae_ref[...] = p
    m_src = jnp.max(asd[:, 0:1])
    m_dst = jnp.max(asd[:, 1:2])
    m_e1 = jnp.max(p[:, 0:8])
    m_e2 = jnp.max(p[:, 8:16])

    @pl.when(i == 0)
    def _():
        mx_s[0] = m_src
        mx_s[1] = m_dst
        mx_s[2] = m_e1
        mx_s[3] = m_e2

    @pl.when(i > 0)
    def _():
        mx_s[0] = jnp.maximum(mx_s[0], m_src)
        mx_s[1] = jnp.maximum(mx_s[1], m_dst)
        mx_s[2] = jnp.maximum(mx_s[2], m_e1)
        mx_s[3] = jnp.maximum(mx_s[3], m_e2)

    @pl.when(i == _NB - 1)
    def _():
        s1 = _lrelu(mx_s[0] + mx_s[1] + mx_s[2])
        s1_ref[...] = jnp.full((8, 128), s1, jnp.float32)
        mx2_ref[...] = jnp.full((8, 128), mx_s[3], jnp.float32)


def _a1_call(x, ea, w, att, ve):
    f32 = jnp.float32
    return pl.pallas_call(
        _a1_body,
        grid=(_NB,),
        in_specs=[
            pl.BlockSpec((_BN, _D), lambda i: (i, 0)),
            pl.BlockSpec((_BEA, _D), lambda i: (i, 0)),
            pl.BlockSpec((_D, _D), lambda i: (0, 0)),
            pl.BlockSpec((_D, _D), lambda i: (0, 0)),
            pl.BlockSpec((_D, _D), lambda i: (0, 0)),
        ],
        out_specs=[
            pl.BlockSpec((_BN, _D), lambda i: (i, 0)),
            pl.BlockSpec((_BN, _D), lambda i: (i, 0)),
            pl.BlockSpec((_BEA, _D), lambda i: (i, 0)),
            pl.BlockSpec((8, 128), lambda i: (0, 0)),
            pl.BlockSpec((8, 128), lambda i: (0, 0)),
        ],
        out_shape=[
            jax.ShapeDtypeStruct((_N, _D), f32),
            jax.ShapeDtypeStruct((_N, _D), f32),
            jax.ShapeDtypeStruct((_E * _DE // _D, _D), f32),
            jax.ShapeDtypeStruct((8, 128), f32),
            jax.ShapeDtypeStruct((8, 128), f32),
        ],
        scratch_shapes=[pltpu.SMEM((4,), f32)],
    )(x, ea, w, att, ve)


def _a2_body(acc_ref, den_ref, b_ref, w_ref, att_ref, mx2_ref,
             h_ref, asd_ref, s2_ref, mx_s):
    i = pl.program_id(0)
    a = acc_ref[0] + acc_ref[1]
    d = den_ref[0, :, 0:1] + den_ref[0, :, 1:2]
    z = jnp.where(d > 0.0, a / d, 0.0) + b_ref[...]
    z = jnp.maximum(z, 0.0)
    hb = z @ w_ref[...]
    h_ref[...] = hb
    asd = hb @ att_ref[...]
    asd_ref[...] = asd
    m_src = jnp.max(asd[:, 0:1])
    m_dst = jnp.max(asd[:, 1:2])

    @pl.when(i == 0)
    def _():
        mx_s[0] = m_src
        mx_s[1] = m_dst

    @pl.when(i > 0)
    def _():
        mx_s[0] = jnp.maximum(mx_s[0], m_src)
        mx_s[1] = jnp.maximum(mx_s[1], m_dst)

    @pl.when(i == _NB - 1)
    def _():
        s2 = _lrelu(mx_s[0] + mx_s[1] + mx2_ref[0, 0])
        s2_ref[...] = jnp.full((8, 128), s2, jnp.float32)


def _a2_call(acc, denr, b, w, att, mx2):
    f32 = jnp.float32
    return pl.pallas_call(
        _a2_body,
        grid=(_NB,),
        in_specs=[
            pl.BlockSpec((2, _BN, _D), lambda i: (0, i, 0)),
            pl.BlockSpec((1, _BN, 2), lambda i: (i, 0, 0)),
            pl.BlockSpec((1, _D), lambda i: (0, 0)),
            pl.BlockSpec((_D, _D), lambda i: (0, 0)),
            pl.BlockSpec((_D, _D), lambda i: (0, 0)),
            pl.BlockSpec((8, 128), lambda i: (0, 0)),
        ],
        out_specs=[
            pl.BlockSpec((_BN, _D), lambda i: (i, 0)),
            pl.BlockSpec((_BN, _D), lambda i: (i, 0)),
            pl.BlockSpec((8, 128), lambda i: (0, 0)),
        ],
        out_shape=[
            jax.ShapeDtypeStruct((_N, _D), f32),
            jax.ShapeDtypeStruct((_N, _D), f32),
            jax.ShapeDtypeStruct((8, 128), f32),
        ],
        scratch_shapes=[pltpu.SMEM((4,), f32)],
    )(acc, denr, b, w, att, mx2)


def _e2_body(acc_ref, den_ref, b_ref, g_ref, be_ref, o_ref):
    a = acc_ref[0] + acc_ref[1]
    d = den_ref[0, :, 0:1] + den_ref[0, :, 1:2]
    o = jnp.where(d > 0.0, a / d, 0.0) + b_ref[...]
    mu = jnp.mean(o, axis=1, keepdims=True)
    v = jnp.mean((o - mu) * (o - mu), axis=1, keepdims=True)
    o_ref[...] = (o - mu) * lax.rsqrt(v + 1e-5) * g_ref[...] + be_ref[...]


def _e2_call(acc, denr, b, g, be):
    return pl.pallas_call(
        _e2_body,
        grid=(_NB,),
        in_specs=[
            pl.BlockSpec((2, _BN, _D), lambda i: (0, i, 0)),
            pl.BlockSpec((1, _BN, 2), lambda i: (i, 0, 0)),
            pl.BlockSpec((1, _D), lambda i: (0, 0)),
            pl.BlockSpec((1, _D), lambda i: (0, 0)),
            pl.BlockSpec((1, _D), lambda i: (0, 0)),
        ],
        out_specs=pl.BlockSpec((_BN, _D), lambda i: (i, 0)),
        out_shape=jax.ShapeDtypeStruct((_N, _D), jnp.float32),
    )(acc, denr, b, g, be)


def _sc_body(h_hbm, sd_hbm, ae_hbm, asrc_hbm, adst_hbm, s_hbm,
             acc_hbm, den_hbm,
             sdix0, sdix1, s_v, buf0, buf1, zden, asb, adb, aeb, exb,
             acc_sp, den_sp, gsem0, gsem1, ssem0, ssem1, sem2, sem3, sem4):
    c = lax.axis_index("c")
    s = lax.axis_index("s")
    wid = c * 16 + s

    pltpu.sync_copy(s_hbm, s_v)

    def _zrow(r, carry):
        for j in range(8):
            buf0[r, pl.ds(j * 16, 16)] = jnp.zeros((16,), jnp.float32)
        return carry

    lax.fori_loop(0, _CH, _zrow, 0)
    base = s * _RPT
    for k in range(7):
        pltpu.sync_copy(buf0, acc_sp.at[pl.ds(base + k * _CH, _CH)])
    pltpu.sync_copy(buf0.at[pl.ds(0, _RPT - 7 * _CH)],
                    acc_sp.at[pl.ds(base + 7 * _CH, _RPT - 7 * _CH)])

    @pl.when(s == 0)
    def _():
        def _zd(r, carry):
            zden[pl.ds(r * 16, 16)] = jnp.zeros((16,), jnp.float32)
            return carry

        lax.fori_loop(0, 125, _zd, 0)
        for k in range(5):
            pltpu.sync_copy(zden, den_sp.at[pl.ds(k * 2000, 2000)])

    plsc.subcore_barrier()

    sv = s_v[0, pl.ds(0, 16)]

    def _exphase(ci, sdixb):
        e1 = pltpu.async_copy(asrc_hbm.at[sdixb.at[0]], asb, sem2)
        e2 = pltpu.async_copy(adst_hbm.at[sdixb.at[1]], adb, sem3)
        pltpu.sync_copy(ae_hbm.at[wid, ci], aeb)
        e1.wait()
        e2.wait()
        for v in range(5):
            sl = pl.ds(v * 16, 16)
            t = asb[sl] + adb[sl] + aeb[sl]
            t = jnp.where(t >= 0.0, t, 0.2 * t)
            exb[sl] = jnp.exp(t - sv)
        pltpu.async_copy(exb, den_sp.at[sdixb.at[1]], sem4, add=True).wait()

    def _scale(bufb):
        def _body(g, c2):
            ex16 = exb[pl.ds(g * 16, 16)]
            for r2 in range(16):
                row = g * 16 + r2
                exs = ex16[r2]
                for j in range(8):
                    sl = pl.ds(j * 16, 16)
                    bufb[row, sl] = bufb[row, sl] * exs
            return c2

        lax.fori_loop(0, _CH // 16, _body, 0)

    bufs = (buf0, buf1)
    sdixs = (sdix0, sdix1)
    gsems = (gsem0, gsem1)
    ssems = (ssem0, ssem1)

    pltpu.sync_copy(sd_hbm.at[wid, c * 0], sdix0)
    pltpu.async_copy(h_hbm.at[sdix0.at[0]], buf0, gsem0)

    def _outer(o, carry):
        for b in range(2):
            bufb, sdixb, gsemb, ssemb = bufs[b], sdixs[b], gsems[b], ssems[b]
            bufo, sdixo, gsemo, ssemo = (bufs[1 - b], sdixs[1 - b],
                                         gsems[1 - b], ssems[1 - b])
            ci = o * 2 + b

            @pl.when(ci < _NC)
            def _():
                @pl.when(ci >= 1)
                def _():
                    pltpu.make_async_copy(
                        bufo, acc_sp.at[sdixo.at[1]], ssemo).wait()

                @pl.when(ci < _NC - 1)
                def _():
                    pltpu.sync_copy(sd_hbm.at[wid, ci + 1], sdixo)
                    pltpu.async_copy(h_hbm.at[sdixo.at[0]], bufo, gsemo)

                _exphase(ci, sdixb)
                pltpu.make_async_copy(
                    h_hbm.at[sdixb.at[0]], bufb, gsemb).wait()
                _scale(bufb)
                pltpu.async_copy(bufb, acc_sp.at[sdixb.at[1]], ssemb,
                                 add=True)
        return carry

    lax.fori_loop(0, (_NC + 1) // 2, _outer, 0)
    pltpu.make_async_copy(buf0, acc_sp.at[sdix0.at[1]], ssem0).wait()

    plsc.subcore_barrier()

    pltpu.sync_copy(acc_sp.at[pl.ds(base, _RPT)], acc_hbm.at[c, s])

    @pl.when(s == 0)
    def _():
        pltpu.sync_copy(den_sp, den_hbm.at[c])


def _sc_call(h, sd4, ae3, asrc, adst, sarr):
    f32 = jnp.float32
    mesh = plsc.VectorSubcoreMesh(core_axis_name="c", subcore_axis_name="s")
    fn = pl.kernel(
        _sc_body,
        out_type=[
            jax.ShapeDtypeStruct((2, 16, _RPT, _D), f32),
            jax.ShapeDtypeStruct((2, _N), f32),
        ],
        mesh=mesh,
        compiler_params=pltpu.CompilerParams(needs_layout_passes=False),
        scratch_types=[
            pltpu.VMEM((2, _CH), jnp.int32),
            pltpu.VMEM((2, _CH), jnp.int32),
            pltpu.VMEM((8, 128), f32),
            pltpu.VMEM((_CH, _D), f32),
            pltpu.VMEM((_CH, _D), f32),
            pltpu.VMEM((2000,), f32),
            pltpu.VMEM((_CH,), f32),
            pltpu.VMEM((_CH,), f32),
            pltpu.VMEM((_CH,), f32),
            pltpu.VMEM((_CH,), f32),
            pltpu.VMEM_SHARED((_N, _D), f32),
            pltpu.VMEM_SHARED((_N,), f32),
            pltpu.SemaphoreType.DMA,
            pltpu.SemaphoreType.DMA,
            pltpu.SemaphoreType.DMA,
            pltpu.SemaphoreType.DMA,
            pltpu.SemaphoreType.DMA,
            pltpu.SemaphoreType.DMA,
            pltpu.SemaphoreType.DMA,
        ],
    )
    acc, den = fn(h, sd4, ae3, asrc, adst, sarr)
    return acc.reshape(2, _N, _D), den


@jax.jit
def kernel(x, edge_index, edge_attr, l1_W, l1_att_src, l1_att_dst, l1_We,
           l1_att_edge, l1_b, l2_W, l2_att_src, l2_att_dst, l2_We,
           l2_att_edge, l2_b, ln_gamma, ln_beta):
    f32 = jnp.float32
    src3 = edge_index[0].reshape(_NT, _NC, _CH)
    dst3 = edge_index[1].reshape(_NT, _NC, _CH)
    sd4 = jnp.stack([src3, dst3], axis=2)
    ea = edge_attr.reshape(_E * _DE // _D, _D)

    ve1 = l1_We @ l1_att_edge
    ve2 = l2_We @ l2_att_edge
    m1 = jnp.kron(jnp.eye(8, dtype=f32), ve1.reshape(_DE, 1))
    m2 = jnp.kron(jnp.eye(8, dtype=f32), ve2.reshape(_DE, 1))
    ve = jnp.concatenate([m1, m2, jnp.zeros((_D, _D - 16), f32)], axis=1)

    att1 = jnp.zeros((_D, _D), f32).at[:, 0].set(l1_att_src).at[:, 1].set(l1_att_dst)
    att2 = jnp.zeros((_D, _D), f32).at[:, 0].set(l2_att_src).at[:, 1].set(l2_att_dst)

    h1, asd1, aep, s1, mx2 = _a1_call(x, ea, l1_W, att1, ve)
    ae1 = aep[:, 0:8].reshape(_NT, _NC, _CH)
    ae2 = aep[:, 8:16].reshape(_NT, _NC, _CH)

    acc1, den1 = _sc_call(h1, sd4, ae1, asd1[:, 0], asd1[:, 1], s1)
    denr1 = den1.T.reshape(_NB, _BN, 2)

    h2, asd2, s2 = _a2_call(acc1, denr1, l1_b.reshape(1, _D), l2_W, att2, mx2)

    acc2, den2 = _sc_call(h2, sd4, ae2, asd2[:, 0], asd2[:, 1], s2)
    denr2 = den2.T.reshape(_NB, _BN, 2)

    return _e2_call(acc2, denr2, l2_b.reshape(1, _D),
                    ln_gamma.reshape(1, _D), ln_beta.reshape(1, _D))


# speedup vs baseline: 30.6649x; 1.3025x over previous
import functools

import jax
import jax.numpy as jnp
from jax import lax
from jax.experimental import pallas as pl
from jax.experimental.pallas import tpu as pltpu
from jax.experimental.pallas import tpu_sc as plsc

_N = 10000
_E = 320000
_D = 128
_DE = 16

_NB = 10
_BN = _N // _NB
_BEA = (_E * _DE // _D) // _NB

_NT = 32
_EW = _E // _NT
_CH = 80
_NC = _EW // _CH
_RPT = _N // 16


def _lrelu(t):
    return jnp.where(t >= 0.0, t, 0.2 * t)


def _a1_body(x_ref, ea_ref, w_ref, att_ref, ve_ref,
             h_ref, asd_ref, ae_ref, s1_ref, mx2_ref, mx_s):
    i = pl.program_id(0)
    hb = x_ref[...] @ w_ref[...]
    h_ref[...] = hb
    asd = hb @ att_ref[...]
    asd_ref[...] = asd
    p = ea_ref[...] @ ve_ref[...]
    ae_ref[...] = p
    m_src = jnp.max(asd[:, 0:1])
    m_dst = jnp.max(asd[:, 1:2])
    m_e1 = jnp.max(p[:, 0:8])
    m_e2 = jnp.max(p[:, 8:16])

    @pl.when(i == 0)
    def _():
        mx_s[0] = m_src
        mx_s[1] = m_dst
        mx_s[2] = m_e1
        mx_s[3] = m_e2

    @pl.when(i > 0)
    def _():
        mx_s[0] = jnp.maximum(mx_s[0], m_src)
        mx_s[1] = jnp.maximum(mx_s[1], m_dst)
        mx_s[2] = jnp.maximum(mx_s[2], m_e1)
        mx_s[3] = jnp.maximum(mx_s[3], m_e2)

    @pl.when(i == _NB - 1)
    def _():
        s1 = _lrelu(mx_s[0] + mx_s[1] + mx_s[2])
        s1_ref[...] = jnp.full((8, 128), s1, jnp.float32)
        mx2_ref[...] = jnp.full((8, 128), mx_s[3], jnp.float32)


def _a1_call(x, ea, w, att, ve):
    f32 = jnp.float32
    return pl.pallas_call(
        _a1_body,
        grid=(_NB,),
        in_specs=[
            pl.BlockSpec((_BN, _D), lambda i: (i, 0)),
            pl.BlockSpec((_BEA, _D), lambda i: (i, 0)),
            pl.BlockSpec((_D, _D), lambda i: (0, 0)),
            pl.BlockSpec((_D, _D), lambda i: (0, 0)),
            pl.BlockSpec((_D, _D), lambda i: (0, 0)),
        ],
        out_specs=[
            pl.BlockSpec((_BN, _D), lambda i: (i, 0)),
            pl.BlockSpec((_BN, _D), lambda i: (i, 0)),
            pl.BlockSpec((_BEA, _D), lambda i: (i, 0)),
            pl.BlockSpec((8, 128), lambda i: (0, 0)),
            pl.BlockSpec((8, 128), lambda i: (0, 0)),
        ],
        out_shape=[
            jax.ShapeDtypeStruct((_N, _D), f32),
            jax.ShapeDtypeStruct((_N, _D), f32),
            jax.ShapeDtypeStruct((_E * _DE // _D, _D), f32),
            jax.ShapeDtypeStruct((8, 128), f32),
            jax.ShapeDtypeStruct((8, 128), f32),
        ],
        scratch_shapes=[pltpu.SMEM((4,), f32)],
    )(x, ea, w, att, ve)


def _a2_body(acc_ref, den_ref, b_ref, w_ref, att_ref, mx2_ref,
             h_ref, asd_ref, s2_ref, mx_s):
    i = pl.program_id(0)
    a = acc_ref[0] + acc_ref[1]
    d = den_ref[0, :, 0:1] + den_ref[0, :, 1:2]
    z = jnp.where(d > 0.0, a / d, 0.0) + b_ref[...]
    z = jnp.maximum(z, 0.0)
    hb = z @ w_ref[...]
    h_ref[...] = hb
    asd = hb @ att_ref[...]
    asd_ref[...] = asd
    m_src = jnp.max(asd[:, 0:1])
    m_dst = jnp.max(asd[:, 1:2])

    @pl.when(i == 0)
    def _():
        mx_s[0] = m_src
        mx_s[1] = m_dst

    @pl.when(i > 0)
    def _():
        mx_s[0] = jnp.maximum(mx_s[0], m_src)
        mx_s[1] = jnp.maximum(mx_s[1], m_dst)

    @pl.when(i == _NB - 1)
    def _():
        s2 = _lrelu(mx_s[0] + mx_s[1] + mx2_ref[0, 0])
        s2_ref[...] = jnp.full((8, 128), s2, jnp.float32)


def _a2_call(acc, denr, b, w, att, mx2):
    f32 = jnp.float32
    return pl.pallas_call(
        _a2_body,
        grid=(_NB,),
        in_specs=[
            pl.BlockSpec((2, _BN, _D), lambda i: (0, i, 0)),
            pl.BlockSpec((1, _BN, 2), lambda i: (i, 0, 0)),
            pl.BlockSpec((1, _D), lambda i: (0, 0)),
            pl.BlockSpec((_D, _D), lambda i: (0, 0)),
            pl.BlockSpec((_D, _D), lambda i: (0, 0)),
            pl.BlockSpec((8, 128), lambda i: (0, 0)),
        ],
        out_specs=[
            pl.BlockSpec((_BN, _D), lambda i: (i, 0)),
            pl.BlockSpec((_BN, _D), lambda i: (i, 0)),
            pl.BlockSpec((8, 128), lambda i: (0, 0)),
        ],
        out_shape=[
            jax.ShapeDtypeStruct((_N, _D), f32),
            jax.ShapeDtypeStruct((_N, _D), f32),
            jax.ShapeDtypeStruct((8, 128), f32),
        ],
        scratch_shapes=[pltpu.SMEM((4,), f32)],
    )(acc, denr, b, w, att, mx2)


def _e2_body(acc_ref, den_ref, b_ref, g_ref, be_ref, o_ref):
    a = acc_ref[0] + acc_ref[1]
    d = den_ref[0, :, 0:1] + den_ref[0, :, 1:2]
    o = jnp.where(d > 0.0, a / d, 0.0) + b_ref[...]
    mu = jnp.mean(o, axis=1, keepdims=True)
    v = jnp.mean((o - mu) * (o - mu), axis=1, keepdims=True)
    o_ref[...] = (o - mu) * lax.rsqrt(v + 1e-5) * g_ref[...] + be_ref[...]


def _e2_call(acc, denr, b, g, be):
    return pl.pallas_call(
        _e2_body,
        grid=(_NB,),
        in_specs=[
            pl.BlockSpec((2, _BN, _D), lambda i: (0, i, 0)),
            pl.BlockSpec((1, _BN, 2), lambda i: (i, 0, 0)),
            pl.BlockSpec((1, _D), lambda i: (0, 0)),
            pl.BlockSpec((1, _D), lambda i: (0, 0)),
            pl.BlockSpec((1, _D), lambda i: (0, 0)),
        ],
        out_specs=pl.BlockSpec((_BN, _D), lambda i: (i, 0)),
        out_shape=jax.ShapeDtypeStruct((_N, _D), jnp.float32),
    )(acc, denr, b, g, be)


def _sc_body(h_hbm, sd_hbm, ae_hbm, asrc_hbm, adst_hbm, s_hbm,
             acc_hbm, den_hbm,
             sdix0, sdix1, s_v, buf0, buf1, zden,
             asb0, asb1, adb0, adb1, aeb0, aeb1, exb0, exb1,
             acc_sp, den_sp,
             gsem0, gsem1, ssem0, ssem1,
             asem0, asem1, bsem0, bsem1, esem0, esem1, dsem0, dsem1):
    c = lax.axis_index("c")
    s = lax.axis_index("s")
    wid = c * 16 + s

    pltpu.sync_copy(s_hbm, s_v)

    def _zrow(r, carry):
        for j in range(8):
            buf0[r, pl.ds(j * 16, 16)] = jnp.zeros((16,), jnp.float32)
        return carry

    lax.fori_loop(0, _CH, _zrow, 0)
    base = s * _RPT
    for k in range(7):
        pltpu.sync_copy(buf0, acc_sp.at[pl.ds(base + k * _CH, _CH)])
    pltpu.sync_copy(buf0.at[pl.ds(0, _RPT - 7 * _CH)],
                    acc_sp.at[pl.ds(base + 7 * _CH, _RPT - 7 * _CH)])

    @pl.when(s == 0)
    def _():
        def _zd(r, carry):
            zden[pl.ds(r * 16, 16)] = jnp.zeros((16,), jnp.float32)
            return carry

        lax.fori_loop(0, 125, _zd, 0)
        for k in range(5):
            pltpu.sync_copy(zden, den_sp.at[pl.ds(k * 2000, 2000)])

    plsc.subcore_barrier()

    sv = s_v[0, pl.ds(0, 16)]

    bufs = (buf0, buf1)
    sdixs = (sdix0, sdix1)
    asbs, adbs, aebs, exbs = (asb0, asb1), (adb0, adb1), (aeb0, aeb1), \
        (exb0, exb1)
    gsems, ssems = (gsem0, gsem1), (ssem0, ssem1)
    asems, bsems, esems, dsems = (asem0, asem1), (bsem0, bsem1), \
        (esem0, esem1), (dsem0, dsem1)

    def _issue_chunk(ci, k):
        pltpu.sync_copy(sd_hbm.at[wid, ci], sdixs[k])
        pltpu.async_copy(h_hbm.at[sdixs[k].at[0]], bufs[k], gsems[k])
        pltpu.async_copy(asrc_hbm.at[sdixs[k].at[0]], asbs[k], asems[k])
        pltpu.async_copy(adst_hbm.at[sdixs[k].at[1]], adbs[k], bsems[k])
        pltpu.async_copy(ae_hbm.at[wid, ci], aebs[k], esems[k])

    def _compute_ex(ci, k):
        pltpu.make_async_copy(
            asrc_hbm.at[sdixs[k].at[0]], asbs[k], asems[k]).wait()
        pltpu.make_async_copy(
            adst_hbm.at[sdixs[k].at[1]], adbs[k], bsems[k]).wait()
        pltpu.make_async_copy(ae_hbm.at[wid, ci], aebs[k], esems[k]).wait()
        for v in range(5):
            sl = pl.ds(v * 16, 16)
            t = asbs[k][sl] + adbs[k][sl] + aebs[k][sl]
            t = jnp.where(t >= 0.0, t, 0.2 * t)
            exbs[k][sl] = jnp.exp(t - sv)
        pltpu.async_copy(exbs[k], den_sp.at[sdixs[k].at[1]], dsems[k],
                         add=True)

    def _scale(bufb, exb):
        def _body(g, c2):
            ex16 = exb[pl.ds(g * 16, 16)]
            for r2 in range(16):
                row = g * 16 + r2
                exs = ex16[r2]
                for j in range(8):
                    sl = pl.ds(j * 16, 16)
                    bufb[row, sl] = bufb[row, sl] * exs
            return c2

        lax.fori_loop(0, _CH // 16, _body, 0)

    _issue_chunk(c * 0, 0)
    _compute_ex(c * 0, 0)

    def _outer(o, carry):
        for b in range(2):
            bo = 1 - b
            ci = o * 2 + b

            @pl.when(ci < _NC)
            def _():
                @pl.when(ci >= 1)
                def _():
                    pltpu.make_async_copy(
                        bufs[bo], acc_sp.at[sdixs[bo].at[1]],
                        ssems[bo]).wait()
                    pltpu.make_async_copy(
                        exbs[bo], den_sp.at[sdixs[bo].at[1]],
                        dsems[bo]).wait()

                @pl.when(ci < _NC - 1)
                def _():
                    _issue_chunk(ci + 1, bo)

                pltpu.make_async_copy(
                    h_hbm.at[sdixs[b].at[0]], bufs[b], gsems[b]).wait()
                _scale(bufs[b], exbs[b])
                pltpu.async_copy(bufs[b], acc_sp.at[sdixs[b].at[1]],
                                 ssems[b], add=True)

                @pl.when(ci < _NC - 1)
                def _():
                    _compute_ex(ci + 1, bo)
        return carry

    lax.fori_loop(0, (_NC + 1) // 2, _outer, 0)
    pltpu.make_async_copy(buf0, acc_sp.at[sdix0.at[1]], ssem0).wait()
    pltpu.make_async_copy(exb0, den_sp.at[sdix0.at[1]], dsem0).wait()

    plsc.subcore_barrier()

    pltpu.sync_copy(acc_sp.at[pl.ds(base, _RPT)], acc_hbm.at[c, s])

    @pl.when(s == 0)
    def _():
        pltpu.sync_copy(den_sp, den_hbm.at[c])


def _sc_call(h, sd4, ae3, asrc, adst, sarr):
    f32 = jnp.float32
    mesh = plsc.VectorSubcoreMesh(core_axis_name="c", subcore_axis_name="s")
    fn = pl.kernel(
        _sc_body,
        out_type=[
            jax.ShapeDtypeStruct((2, 16, _RPT, _D), f32),
            jax.ShapeDtypeStruct((2, _N), f32),
        ],
        mesh=mesh,
        compiler_params=pltpu.CompilerParams(needs_layout_passes=False),
        scratch_types=[
            pltpu.VMEM((2, _CH), jnp.int32),
            pltpu.VMEM((2, _CH), jnp.int32),
            pltpu.VMEM((8, 128), f32),
            pltpu.VMEM((_CH, _D), f32),
            pltpu.VMEM((_CH, _D), f32),
            pltpu.VMEM((2000,), f32),
        ] + [pltpu.VMEM((_CH,), f32)] * 8
        + [
            pltpu.VMEM_SHARED((_N, _D), f32),
            pltpu.VMEM_SHARED((_N,), f32),
        ] + [pltpu.SemaphoreType.DMA] * 12,
    )
    acc, den = fn(h, sd4, ae3, asrc, adst, sarr)
    return acc.reshape(2, _N, _D), den


@jax.jit
def kernel(x, edge_index, edge_attr, l1_W, l1_att_src, l1_att_dst, l1_We,
           l1_att_edge, l1_b, l2_W, l2_att_src, l2_att_dst, l2_We,
           l2_att_edge, l2_b, ln_gamma, ln_beta):
    f32 = jnp.float32
    src3 = edge_index[0].reshape(_NT, _NC, _CH)
    dst3 = edge_index[1].reshape(_NT, _NC, _CH)
    sd4 = jnp.stack([src3, dst3], axis=2)
    ea = edge_attr.reshape(_E * _DE // _D, _D)

    ve1 = l1_We @ l1_att_edge
    ve2 = l2_We @ l2_att_edge
    m1 = jnp.kron(jnp.eye(8, dtype=f32), ve1.reshape(_DE, 1))
    m2 = jnp.kron(jnp.eye(8, dtype=f32), ve2.reshape(_DE, 1))
    ve = jnp.concatenate([m1, m2, jnp.zeros((_D, _D - 16), f32)], axis=1)

    att1 = jnp.zeros((_D, _D), f32).at[:, 0].set(l1_att_src).at[:, 1].set(l1_att_dst)
    att2 = jnp.zeros((_D, _D), f32).at[:, 0].set(l2_att_src).at[:, 1].set(l2_att_dst)

    h1, asd1, aep, s1, mx2 = _a1_call(x, ea, l1_W, att1, ve)
    ae1 = aep[:, 0:8].reshape(_NT, _NC, _CH)
    ae2 = aep[:, 8:16].reshape(_NT, _NC, _CH)

    acc1, den1 = _sc_call(h1, sd4, ae1, asd1[:, 0], asd1[:, 1], s1)
    denr1 = den1.T.reshape(_NB, _BN, 2)

    h2, asd2, s2 = _a2_call(acc1, denr1, l1_b.reshape(1, _D), l2_W, att2, mx2)

    acc2, den2 = _sc_call(h2, sd4, ae2, asd2[:, 0], asd2[:, 1], s2)
    denr2 = den2.T.reshape(_NB, _BN, 2)

    return _e2_call(acc2, denr2, l2_b.reshape(1, _D),
                    ln_gamma.reshape(1, _D), ln_beta.reshape(1, _D))


# speedup vs baseline: 35.5460x; 1.1592x over previous
import functools

import jax
import jax.numpy as jnp
from jax import lax
from jax.experimental import pallas as pl
from jax.experimental.pallas import tpu as pltpu
from jax.experimental.pallas import tpu_sc as plsc

_N = 10000
_E = 320000
_D = 128
_DE = 16

_NB = 10
_BN = _N // _NB
_BEA = (_E * _DE // _D) // _NB

_NT = 32
_EW = _E // _NT
_CH = 80
_NC = _EW // _CH
_RPT = _N // 16


def _lrelu(t):
    return jnp.where(t >= 0.0, t, 0.2 * t)


def _a1_body(x_ref, ea_ref, w_ref, att_ref, ve_ref,
             h_ref, asd_ref, ae_ref, s1_ref, mx2_ref, mx_s):
    i = pl.program_id(0)
    hb = x_ref[...] @ w_ref[...]
    h_ref[...] = hb
    asd = hb @ att_ref[...]
    asd_ref[...] = asd
    p = ea_ref[...] @ ve_ref[...]
    ae_ref[...] = p
    m_src = jnp.max(asd[:, 0:1])
    m_dst = jnp.max(asd[:, 1:2])
    m_e1 = jnp.max(p[:, 0:8])
    m_e2 = jnp.max(p[:, 8:16])

    @pl.when(i == 0)
    def _():
        mx_s[0] = m_src
        mx_s[1] = m_dst
        mx_s[2] = m_e1
        mx_s[3] = m_e2

    @pl.when(i > 0)
    def _():
        mx_s[0] = jnp.maximum(mx_s[0], m_src)
        mx_s[1] = jnp.maximum(mx_s[1], m_dst)
        mx_s[2] = jnp.maximum(mx_s[2], m_e1)
        mx_s[3] = jnp.maximum(mx_s[3], m_e2)

    @pl.when(i == _NB - 1)
    def _():
        s1 = _lrelu(mx_s[0] + mx_s[1] + mx_s[2])
        s1_ref[...] = jnp.full((8, 128), s1, jnp.float32)
        mx2_ref[...] = jnp.full((8, 128), mx_s[3], jnp.float32)


def _a1_call(x, ea, w, att, ve):
    f32 = jnp.float32
    return pl.pallas_call(
        _a1_body,
        grid=(_NB,),
        in_specs=[
            pl.BlockSpec((_BN, _D), lambda i: (i, 0)),
            pl.BlockSpec((_BEA, _D), lambda i: (i, 0)),
            pl.BlockSpec((_D, _D), lambda i: (0, 0)),
            pl.BlockSpec((_D, _D), lambda i: (0, 0)),
            pl.BlockSpec((_D, _D), lambda i: (0, 0)),
        ],
        out_specs=[
            pl.BlockSpec((_BN, _D), lambda i: (i, 0)),
            pl.BlockSpec((_BN, _D), lambda i: (i, 0)),
            pl.BlockSpec((_BEA, _D), lambda i: (i, 0)),
            pl.BlockSpec((8, 128), lambda i: (0, 0)),
            pl.BlockSpec((8, 128), lambda i: (0, 0)),
        ],
        out_shape=[
            jax.ShapeDtypeStruct((_N, _D), f32),
            jax.ShapeDtypeStruct((_N, _D), f32),
            jax.ShapeDtypeStruct((_E * _DE // _D, _D), f32),
            jax.ShapeDtypeStruct((8, 128), f32),
            jax.ShapeDtypeStruct((8, 128), f32),
        ],
        scratch_shapes=[pltpu.SMEM((4,), f32)],
    )(x, ea, w, att, ve)


def _a2_body(acc_ref, den_ref, b_ref, w_ref, att_ref, mx2_ref,
             h_ref, asd_ref, s2_ref, mx_s):
    i = pl.program_id(0)
    a = acc_ref[0] + acc_ref[1]
    d = den_ref[0, :, 0:1] + den_ref[0, :, 1:2]
    z = jnp.where(d > 0.0, a / d, 0.0) + b_ref[...]
    z = jnp.maximum(z, 0.0)
    hb = z @ w_ref[...]
    h_ref[...] = hb
    asd = hb @ att_ref[...]
    asd_ref[...] = asd
    m_src = jnp.max(asd[:, 0:1])
    m_dst = jnp.max(asd[:, 1:2])

    @pl.when(i == 0)
    def _():
        mx_s[0] = m_src
        mx_s[1] = m_dst

    @pl.when(i > 0)
    def _():
        mx_s[0] = jnp.maximum(mx_s[0], m_src)
        mx_s[1] = jnp.maximum(mx_s[1], m_dst)

    @pl.when(i == _NB - 1)
    def _():
        s2 = _lrelu(mx_s[0] + mx_s[1] + mx2_ref[0, 0])
        s2_ref[...] = jnp.full((8, 128), s2, jnp.float32)


def _a2_call(acc, denr, b, w, att, mx2):
    f32 = jnp.float32
    return pl.pallas_call(
        _a2_body,
        grid=(_NB,),
        in_specs=[
            pl.BlockSpec((2, _BN, _D), lambda i: (0, i, 0)),
            pl.BlockSpec((1, _BN, 2), lambda i: (i, 0, 0)),
            pl.BlockSpec((1, _D), lambda i: (0, 0)),
            pl.BlockSpec((_D, _D), lambda i: (0, 0)),
            pl.BlockSpec((_D, _D), lambda i: (0, 0)),
            pl.BlockSpec((8, 128), lambda i: (0, 0)),
        ],
        out_specs=[
            pl.BlockSpec((_BN, _D), lambda i: (i, 0)),
            pl.BlockSpec((_BN, _D), lambda i: (i, 0)),
            pl.BlockSpec((8, 128), lambda i: (0, 0)),
        ],
        out_shape=[
            jax.ShapeDtypeStruct((_N, _D), f32),
            jax.ShapeDtypeStruct((_N, _D), f32),
            jax.ShapeDtypeStruct((8, 128), f32),
        ],
        scratch_shapes=[pltpu.SMEM((4,), f32)],
    )(acc, denr, b, w, att, mx2)


def _e2_body(acc_ref, den_ref, b_ref, g_ref, be_ref, o_ref):
    a = acc_ref[0] + acc_ref[1]
    d = den_ref[0, :, 0:1] + den_ref[0, :, 1:2]
    o = jnp.where(d > 0.0, a / d, 0.0) + b_ref[...]
    mu = jnp.mean(o, axis=1, keepdims=True)
    v = jnp.mean((o - mu) * (o - mu), axis=1, keepdims=True)
    o_ref[...] = (o - mu) * lax.rsqrt(v + 1e-5) * g_ref[...] + be_ref[...]


def _e2_call(acc, denr, b, g, be):
    return pl.pallas_call(
        _e2_body,
        grid=(_NB,),
        in_specs=[
            pl.BlockSpec((2, _BN, _D), lambda i: (0, i, 0)),
            pl.BlockSpec((1, _BN, 2), lambda i: (i, 0, 0)),
            pl.BlockSpec((1, _D), lambda i: (0, 0)),
            pl.BlockSpec((1, _D), lambda i: (0, 0)),
            pl.BlockSpec((1, _D), lambda i: (0, 0)),
        ],
        out_specs=pl.BlockSpec((_BN, _D), lambda i: (i, 0)),
        out_shape=jax.ShapeDtypeStruct((_N, _D), jnp.float32),
    )(acc, denr, b, g, be)


def _sc_body(h_hbm, sd_hbm, asrc_hbm, adst_hbm, s_hbm,
             acc_hbm, den_hbm,
             sdix0, sdix1, sdix2, sdix3, sdix4, sdix5,
             s_v, buf0, buf1, buf2, zden,
             asb0, asb1, asb2, adb0, adb1, adb2, exb0, exb1, exb2,
             acc_sp, den_sp,
             gsem0, gsem1, gsem2, ssem0, ssem1, ssem2,
             asem0, asem1, asem2, bsem0, bsem1, bsem2,
             dsem0, dsem1, dsem2,
             isem0, isem1, isem2, isem3, isem4, isem5):
    c = lax.axis_index("c")
    s = lax.axis_index("s")
    wid = c * 16 + s

    pltpu.sync_copy(s_hbm, s_v)

    def _zrow(r, carry):
        for j in range(8):
            buf0[r, pl.ds(j * 16, 16)] = jnp.zeros((16,), jnp.float32)
        return carry

    lax.fori_loop(0, _CH, _zrow, 0)
    base = s * _RPT
    for k in range(7):
        pltpu.sync_copy(buf0, acc_sp.at[pl.ds(base + k * _CH, _CH)])
    pltpu.sync_copy(buf0.at[pl.ds(0, _RPT - 7 * _CH)],
                    acc_sp.at[pl.ds(base + 7 * _CH, _RPT - 7 * _CH)])

    @pl.when(s == 0)
    def _():
        def _zd(r, carry):
            zden[pl.ds(r * 16, 16)] = jnp.zeros((16,), jnp.float32)
            return carry

        lax.fori_loop(0, 125, _zd, 0)
        for k in range(5):
            pltpu.sync_copy(zden, den_sp.at[pl.ds(k * 2000, 2000)])

    plsc.subcore_barrier()

    sv = s_v[0, pl.ds(0, 16)]

    bufs = (buf0, buf1, buf2)
    sdixs = (sdix0, sdix1, sdix2, sdix3, sdix4, sdix5)
    asbs, adbs, exbs = (asb0, asb1, asb2), (adb0, adb1, adb2), \
        (exb0, exb1, exb2)
    gsems, ssems = (gsem0, gsem1, gsem2), (ssem0, ssem1, ssem2)
    asems, bsems, dsems = (asem0, asem1, asem2), (bsem0, bsem1, bsem2), \
        (dsem0, dsem1, dsem2)
    isems = (isem0, isem1, isem2, isem3, isem4, isem5)

    def _issue_gathers(k, m):
        pltpu.async_copy(h_hbm.at[sdixs[m].at[0]], bufs[k], gsems[k])
        pltpu.async_copy(asrc_hbm.at[sdixs[m].at[0]], asbs[k], asems[k])
        pltpu.async_copy(adst_hbm.at[sdixs[m].at[1]], adbs[k], bsems[k])

    pltpu.sync_copy(sd_hbm.at[wid, c * 0], sdix0)
    pltpu.async_copy(sd_hbm.at[wid, c * 0 + 1], sdix1, isem1)
    _issue_gathers(0, 0)

    def _outer(o, carry):
        for b in range(6):
            k = b % 3
            ci = o * 6 + b

            @pl.when(ci < _NC)
            def _():
                sdm = sdixs[b]
                sdm1 = sdixs[(b + 1) % 6]
                sdm2 = sdixs[(b + 2) % 6]
                kn = (b + 1) % 3

                @pl.when(ci >= 2)
                def _():
                    pltpu.make_async_copy(
                        bufs[kn], acc_sp.at[sdm.at[1]], ssems[kn]).wait()
                    pltpu.make_async_copy(
                        exbs[kn], den_sp.at[sdm.at[1]], dsems[kn]).wait()

                @pl.when(ci + 2 < _NC)
                def _():
                    pltpu.async_copy(sd_hbm.at[wid, ci + 2], sdm2,
                                     isems[(b + 2) % 6])

                @pl.when(ci + 1 < _NC)
                def _():
                    pltpu.make_async_copy(
                        sd_hbm.at[wid, ci + 1], sdm1,
                        isems[(b + 1) % 6]).wait()
                    _issue_gathers(kn, (b + 1) % 6)

                pltpu.make_async_copy(
                    asrc_hbm.at[sdm.at[0]], asbs[k], asems[k]).wait()
                pltpu.make_async_copy(
                    adst_hbm.at[sdm.at[1]], adbs[k], bsems[k]).wait()
                for v in range(5):
                    sl = pl.ds(v * 16, 16)
                    ae16 = plsc.bitcast(sdm[2, sl], jnp.float32)
                    t = asbs[k][sl] + adbs[k][sl] + ae16
                    t = jnp.where(t >= 0.0, t, 0.2 * t)
                    exbs[k][sl] = jnp.exp(t - sv)
                pltpu.async_copy(exbs[k], den_sp.at[sdm.at[1]], dsems[k],
                                 add=True)
                pltpu.make_async_copy(
                    h_hbm.at[sdm.at[0]], bufs[k], gsems[k]).wait()

                def _body(g, c2):
                    ex16 = exbs[k][pl.ds(g * 16, 16)]
                    for r2 in range(16):
                        row = g * 16 + r2
                        exs = ex16[r2]
                        for j in range(8):
                            sl = pl.ds(j * 16, 16)
                            bufs[k][row, sl] = bufs[k][row, sl] * exs
                    return c2

                lax.fori_loop(0, _CH // 16, _body, 0)
                pltpu.async_copy(bufs[k], acc_sp.at[sdm.at[1]], ssems[k],
                                 add=True)
        return carry

    lax.fori_loop(0, (_NC + 5) // 6, _outer, 0)
    for k in range(2):
        pltpu.make_async_copy(bufs[k], acc_sp.at[sdixs[k].at[1]],
                              ssems[k]).wait()
        pltpu.make_async_copy(exbs[k], den_sp.at[sdixs[k].at[1]],
                              dsems[k]).wait()

    plsc.subcore_barrier()

    pltpu.sync_copy(acc_sp.at[pl.ds(base, _RPT)], acc_hbm.at[c, s])

    @pl.when(s == 0)
    def _():
        pltpu.sync_copy(den_sp, den_hbm.at[c])


def _sc_call(h, sdae, asrc, adst, sarr):
    f32 = jnp.float32
    mesh = plsc.VectorSubcoreMesh(core_axis_name="c", subcore_axis_name="s")
    fn = pl.kernel(
        _sc_body,
        out_type=[
            jax.ShapeDtypeStruct((2, 16, _RPT, _D), f32),
            jax.ShapeDtypeStruct((2, _N), f32),
        ],
        mesh=mesh,
        compiler_params=pltpu.CompilerParams(needs_layout_passes=False),
        scratch_types=[pltpu.VMEM((3, _CH), jnp.int32)] * 6
        + [
            pltpu.VMEM((8, 128), f32),
            pltpu.VMEM((_CH, _D), f32),
            pltpu.VMEM((_CH, _D), f32),
            pltpu.VMEM((_CH, _D), f32),
            pltpu.VMEM((2000,), f32),
        ] + [pltpu.VMEM((_CH,), f32)] * 9
        + [
            pltpu.VMEM_SHARED((_N, _D), f32),
            pltpu.VMEM_SHARED((_N,), f32),
        ] + [pltpu.SemaphoreType.DMA] * 21,
    )
    acc, den = fn(h, sdae, asrc, adst, sarr)
    return acc.reshape(2, _N, _D), den


@jax.jit
def kernel(x, edge_index, edge_attr, l1_W, l1_att_src, l1_att_dst, l1_We,
           l1_att_edge, l1_b, l2_W, l2_att_src, l2_att_dst, l2_We,
           l2_att_edge, l2_b, ln_gamma, ln_beta):
    f32 = jnp.float32
    src3 = edge_index[0].reshape(_NT, _NC, _CH)
    dst3 = edge_index[1].reshape(_NT, _NC, _CH)
    ea = edge_attr.reshape(_E * _DE // _D, _D)

    ve1 = l1_We @ l1_att_edge
    ve2 = l2_We @ l2_att_edge
    m1 = jnp.kron(jnp.eye(8, dtype=f32), ve1.reshape(_DE, 1))
    m2 = jnp.kron(jnp.eye(8, dtype=f32), ve2.reshape(_DE, 1))
    ve = jnp.concatenate([m1, m2, jnp.zeros((_D, _D - 16), f32)], axis=1)

    att1 = jnp.zeros((_D, _D), f32).at[:, 0].set(l1_att_src).at[:, 1].set(l1_att_dst)
    att2 = jnp.zeros((_D, _D), f32).at[:, 0].set(l2_att_src).at[:, 1].set(l2_att_dst)

    h1, asd1, aep, s1, mx2 = _a1_call(x, ea, l1_W, att1, ve)
    ae1 = lax.bitcast_convert_type(
        aep[:, 0:8].reshape(_NT, _NC, _CH), jnp.int32)
    ae2 = lax.bitcast_convert_type(
        aep[:, 8:16].reshape(_NT, _NC, _CH), jnp.int32)
    sdae1 = jnp.stack([src3, dst3, ae1], axis=2)
    sdae2 = jnp.stack([src3, dst3, ae2], axis=2)

    acc1, den1 = _sc_call(h1, sdae1, asd1[:, 0], asd1[:, 1], s1)
    denr1 = den1.T.reshape(_NB, _BN, 2)

    h2, asd2, s2 = _a2_call(acc1, denr1, l1_b.reshape(1, _D), l2_W, att2, mx2)

    acc2, den2 = _sc_call(h2, sdae2, asd2[:, 0], asd2[:, 1], s2)
    denr2 = den2.T.reshape(_NB, _BN, 2)

    return _e2_call(acc2, denr2, l2_b.reshape(1, _D),
                    ln_gamma.reshape(1, _D), ln_beta.reshape(1, _D))


# speedup vs baseline: 35.5911x; 1.0013x over previous
import functools

import jax
import jax.numpy as jnp
from jax import lax
from jax.experimental import pallas as pl
from jax.experimental.pallas import tpu as pltpu
from jax.experimental.pallas import tpu_sc as plsc

_N = 10000
_E = 320000
_D = 128
_DE = 16

_NB = 10
_BN = _N // _NB
_BEA = (_E * _DE // _D) // _NB

_NT = 32
_EW = _E // _NT
_CH = 80
_NC = _EW // _CH
_RPT = _N // 16


def _lrelu(t):
    return jnp.where(t >= 0.0, t, 0.2 * t)


def _a1_body(x_ref, ea_ref, w_ref, att_ref, ve_ref,
             h_ref, asd_ref, ae_ref, s1_ref, mx2_ref, mx_s):
    i = pl.program_id(0)
    hb = x_ref[...] @ w_ref[...]
    h_ref[...] = hb
    asd = hb @ att_ref[...]
    asd_ref[...] = asd
    p = ea_ref[...] @ ve_ref[...]
    ae_ref[...] = p
    m_src = jnp.max(asd[:, 0:1])
    m_dst = jnp.max(asd[:, 1:2])
    m_e1 = jnp.max(p[:, 0:8])
    m_e2 = jnp.max(p[:, 8:16])

    @pl.when(i == 0)
    def _():
        mx_s[0] = m_src
        mx_s[1] = m_dst
        mx_s[2] = m_e1
        mx_s[3] = m_e2

    @pl.when(i > 0)
    def _():
        mx_s[0] = jnp.maximum(mx_s[0], m_src)
        mx_s[1] = jnp.maximum(mx_s[1], m_dst)
        mx_s[2] = jnp.maximum(mx_s[2], m_e1)
        mx_s[3] = jnp.maximum(mx_s[3], m_e2)

    @pl.when(i == _NB - 1)
    def _():
        s1 = _lrelu(mx_s[0] + mx_s[1] + mx_s[2])
        s1_ref[...] = jnp.full((8, 128), s1, jnp.float32)
        mx2_ref[...] = jnp.full((8, 128), mx_s[3], jnp.float32)


def _a1_call(x, ea, w, att, ve):
    f32 = jnp.float32
    return pl.pallas_call(
        _a1_body,
        grid=(_NB,),
        in_specs=[
            pl.BlockSpec((_BN, _D), lambda i: (i, 0)),
            pl.BlockSpec((_BEA, _D), lambda i: (i, 0)),
            pl.BlockSpec((_D, _D), lambda i: (0, 0)),
            pl.BlockSpec((_D, _D), lambda i: (0, 0)),
            pl.BlockSpec((_D, 16), lambda i: (0, 0)),
        ],
        out_specs=[
            pl.BlockSpec((_BN, _D), lambda i: (i, 0)),
            pl.BlockSpec((_BN, _D), lambda i: (i, 0)),
            pl.BlockSpec((_BEA, 16), lambda i: (i, 0)),
            pl.BlockSpec((8, 128), lambda i: (0, 0)),
            pl.BlockSpec((8, 128), lambda i: (0, 0)),
        ],
        out_shape=[
            jax.ShapeDtypeStruct((_N, _D), f32),
            jax.ShapeDtypeStruct((_N, _D), f32),
            jax.ShapeDtypeStruct((_E * _DE // _D, 16), f32),
            jax.ShapeDtypeStruct((8, 128), f32),
            jax.ShapeDtypeStruct((8, 128), f32),
        ],
        scratch_shapes=[pltpu.SMEM((4,), f32)],
    )(x, ea, w, att, ve)


def _a2_body(acc_ref, den_ref, b_ref, w_ref, att_ref, mx2_ref,
             h_ref, asd_ref, s2_ref, mx_s):
    i = pl.program_id(0)
    a = acc_ref[0] + acc_ref[1]
    d = den_ref[0, :, 0:1] + den_ref[0, :, 1:2]
    z = jnp.where(d > 0.0, a / d, 0.0) + b_ref[...]
    z = jnp.maximum(z, 0.0)
    hb = z @ w_ref[...]
    h_ref[...] = hb
    asd = hb @ att_ref[...]
    asd_ref[...] = asd
    m_src = jnp.max(asd[:, 0:1])
    m_dst = jnp.max(asd[:, 1:2])

    @pl.when(i == 0)
    def _():
        mx_s[0] = m_src
        mx_s[1] = m_dst

    @pl.when(i > 0)
    def _():
        mx_s[0] = jnp.maximum(mx_s[0], m_src)
        mx_s[1] = jnp.maximum(mx_s[1], m_dst)

    @pl.when(i == _NB - 1)
    def _():
        s2 = _lrelu(mx_s[0] + mx_s[1] + mx2_ref[0, 0])
        s2_ref[...] = jnp.full((8, 128), s2, jnp.float32)


def _a2_call(acc, denr, b, w, att, mx2):
    f32 = jnp.float32
    return pl.pallas_call(
        _a2_body,
        grid=(_NB,),
        in_specs=[
            pl.BlockSpec((2, _BN, _D), lambda i: (0, i, 0)),
            pl.BlockSpec((1, _BN, 2), lambda i: (i, 0, 0)),
            pl.BlockSpec((1, _D), lambda i: (0, 0)),
            pl.BlockSpec((_D, _D), lambda i: (0, 0)),
            pl.BlockSpec((_D, _D), lambda i: (0, 0)),
            pl.BlockSpec((8, 128), lambda i: (0, 0)),
        ],
        out_specs=[
            pl.BlockSpec((_BN, _D), lambda i: (i, 0)),
            pl.BlockSpec((_BN, _D), lambda i: (i, 0)),
            pl.BlockSpec((8, 128), lambda i: (0, 0)),
        ],
        out_shape=[
            jax.ShapeDtypeStruct((_N, _D), f32),
            jax.ShapeDtypeStruct((_N, _D), f32),
            jax.ShapeDtypeStruct((8, 128), f32),
        ],
        scratch_shapes=[pltpu.SMEM((4,), f32)],
    )(acc, denr, b, w, att, mx2)


def _e2_body(acc_ref, den_ref, b_ref, g_ref, be_ref, o_ref):
    a = acc_ref[0] + acc_ref[1]
    d = den_ref[0, :, 0:1] + den_ref[0, :, 1:2]
    o = jnp.where(d > 0.0, a / d, 0.0) + b_ref[...]
    mu = jnp.mean(o, axis=1, keepdims=True)
    v = jnp.mean((o - mu) * (o - mu), axis=1, keepdims=True)
    o_ref[...] = (o - mu) * lax.rsqrt(v + 1e-5) * g_ref[...] + be_ref[...]


def _e2_call(acc, denr, b, g, be):
    return pl.pallas_call(
        _e2_body,
        grid=(_NB,),
        in_specs=[
            pl.BlockSpec((2, _BN, _D), lambda i: (0, i, 0)),
            pl.BlockSpec((1, _BN, 2), lambda i: (i, 0, 0)),
            pl.BlockSpec((1, _D), lambda i: (0, 0)),
            pl.BlockSpec((1, _D), lambda i: (0, 0)),
            pl.BlockSpec((1, _D), lambda i: (0, 0)),
        ],
        out_specs=pl.BlockSpec((_BN, _D), lambda i: (i, 0)),
        out_shape=jax.ShapeDtypeStruct((_N, _D), jnp.float32),
    )(acc, denr, b, g, be)


def _sc_body(h_hbm, sd_hbm, asrc_hbm, adst_hbm, s_hbm,
             acc_hbm, den_hbm,
             sdix0, sdix1, sdix2, sdix3, sdix4, sdix5,
             s_v, buf0, buf1, buf2, zden,
             asb0, asb1, asb2, adb0, adb1, adb2, exb0, exb1, exb2,
             acc_sp, den_sp,
             gsem0, gsem1, gsem2, ssem0, ssem1, ssem2,
             asem0, asem1, asem2, bsem0, bsem1, bsem2,
             dsem0, dsem1, dsem2,
             isem0, isem1, isem2, isem3, isem4, isem5):
    c = lax.axis_index("c")
    s = lax.axis_index("s")
    wid = c * 16 + s

    pltpu.sync_copy(s_hbm, s_v)

    def _zrow(r, carry):
        for j in range(8):
            buf0[r, pl.ds(j * 16, 16)] = jnp.zeros((16,), jnp.float32)
        return carry

    lax.fori_loop(0, _CH, _zrow, 0)
    base = s * _RPT
    for k in range(7):
        pltpu.sync_copy(buf0, acc_sp.at[pl.ds(base + k * _CH, _CH)])
    pltpu.sync_copy(buf0.at[pl.ds(0, _RPT - 7 * _CH)],
                    acc_sp.at[pl.ds(base + 7 * _CH, _RPT - 7 * _CH)])

    @pl.when(s == 0)
    def _():
        def _zd(r, carry):
            zden[pl.ds(r * 16, 16)] = jnp.zeros((16,), jnp.float32)
            return carry

        lax.fori_loop(0, 125, _zd, 0)
        for k in range(5):
            pltpu.sync_copy(zden, den_sp.at[pl.ds(k * 2000, 2000)])

    plsc.subcore_barrier()

    sv = s_v[0, pl.ds(0, 16)]

    bufs = (buf0, buf1, buf2)
    sdixs = (sdix0, sdix1, sdix2, sdix3, sdix4, sdix5)
    asbs, adbs, exbs = (asb0, asb1, asb2), (adb0, adb1, adb2), \
        (exb0, exb1, exb2)
    gsems, ssems = (gsem0, gsem1, gsem2), (ssem0, ssem1, ssem2)
    asems, bsems, dsems = (asem0, asem1, asem2), (bsem0, bsem1, bsem2), \
        (dsem0, dsem1, dsem2)
    isems = (isem0, isem1, isem2, isem3, isem4, isem5)

    def _issue_gathers(k, m):
        pltpu.async_copy(h_hbm.at[sdixs[m].at[0]], bufs[k], gsems[k])
        pltpu.async_copy(asrc_hbm.at[sdixs[m].at[0]], asbs[k], asems[k])
        pltpu.async_copy(adst_hbm.at[sdixs[m].at[1]], adbs[k], bsems[k])

    pltpu.sync_copy(sd_hbm.at[wid, c * 0], sdix0)
    pltpu.async_copy(sd_hbm.at[wid, c * 0 + 1], sdix1, isem1)
    _issue_gathers(0, 0)

    def _outer(o, carry):
        for b in range(6):
            k = b % 3
            ci = o * 6 + b

            @pl.when(ci < _NC)
            def _():
                sdm = sdixs[b]
                sdm1 = sdixs[(b + 1) % 6]
                sdm2 = sdixs[(b + 2) % 6]
                kn = (b + 1) % 3

                @pl.when(ci >= 2)
                def _():
                    pltpu.make_async_copy(
                        bufs[kn], acc_sp.at[sdm.at[1]], ssems[kn]).wait()
                    pltpu.make_async_copy(
                        exbs[kn], den_sp.at[sdm.at[1]], dsems[kn]).wait()

                @pl.when(ci + 2 < _NC)
                def _():
                    pltpu.async_copy(sd_hbm.at[wid, ci + 2], sdm2,
                                     isems[(b + 2) % 6])

                @pl.when(ci + 1 < _NC)
                def _():
                    pltpu.make_async_copy(
                        sd_hbm.at[wid, ci + 1], sdm1,
                        isems[(b + 1) % 6]).wait()
                    _issue_gathers(kn, (b + 1) % 6)

                pltpu.make_async_copy(
                    asrc_hbm.at[sdm.at[0]], asbs[k], asems[k]).wait()
                pltpu.make_async_copy(
                    adst_hbm.at[sdm.at[1]], adbs[k], bsems[k]).wait()
                for v in range(5):
                    sl = pl.ds(v * 16, 16)
                    ae16 = plsc.bitcast(sdm[2, sl], jnp.float32)
                    t = asbs[k][sl] + adbs[k][sl] + ae16
                    t = jnp.where(t >= 0.0, t, 0.2 * t)
                    exbs[k][sl] = jnp.exp(t - sv)
                pltpu.async_copy(exbs[k], den_sp.at[sdm.at[1]], dsems[k],
                                 add=True)
                pltpu.make_async_copy(
                    h_hbm.at[sdm.at[0]], bufs[k], gsems[k]).wait()

                def _body(g, c2):
                    ex16 = exbs[k][pl.ds(g * 16, 16)]
                    for r2 in range(16):
                        row = g * 16 + r2
                        exs = ex16[r2]
                        for j in range(8):
                            sl = pl.ds(j * 16, 16)
                            bufs[k][row, sl] = bufs[k][row, sl] * exs
                    return c2

                lax.fori_loop(0, _CH // 16, _body, 0)
                pltpu.async_copy(bufs[k], acc_sp.at[sdm.at[1]], ssems[k],
                                 add=True)
        return carry

    lax.fori_loop(0, (_NC + 5) // 6, _outer, 0)
    for k in range(2):
        pltpu.make_async_copy(bufs[k], acc_sp.at[sdixs[k].at[1]],
                              ssems[k]).wait()
        pltpu.make_async_copy(exbs[k], den_sp.at[sdixs[k].at[1]],
                              dsems[k]).wait()

    plsc.subcore_barrier()

    pltpu.sync_copy(acc_sp.at[pl.ds(base, _RPT)], acc_hbm.at[c, s])

    @pl.when(s == 0)
    def _():
        pltpu.sync_copy(den_sp, den_hbm.at[c])


def _sc_call(h, sdae, asrc, adst, sarr):
    f32 = jnp.float32
    mesh = plsc.VectorSubcoreMesh(core_axis_name="c", subcore_axis_name="s")
    fn = pl.kernel(
        _sc_body,
        out_type=[
            jax.ShapeDtypeStruct((2, 16, _RPT, _D), f32),
            jax.ShapeDtypeStruct((2, _N), f32),
        ],
        mesh=mesh,
        compiler_params=pltpu.CompilerParams(needs_layout_passes=False),
        scratch_types=[pltpu.VMEM((3, _CH), jnp.int32)] * 6
        + [
            pltpu.VMEM((8, 128), f32),
            pltpu.VMEM((_CH, _D), f32),
            pltpu.VMEM((_CH, _D), f32),
            pltpu.VMEM((_CH, _D), f32),
            pltpu.VMEM((2000,), f32),
        ] + [pltpu.VMEM((_CH,), f32)] * 9
        + [
            pltpu.VMEM_SHARED((_N, _D), f32),
            pltpu.VMEM_SHARED((_N,), f32),
        ] + [pltpu.SemaphoreType.DMA] * 21,
    )
    acc, den = fn(h, sdae, asrc, adst, sarr)
    return acc.reshape(2, _N, _D), den


@jax.jit
def kernel(x, edge_index, edge_attr, l1_W, l1_att_src, l1_att_dst, l1_We,
           l1_att_edge, l1_b, l2_W, l2_att_src, l2_att_dst, l2_We,
           l2_att_edge, l2_b, ln_gamma, ln_beta):
    f32 = jnp.float32
    src3 = edge_index[0].reshape(_NT, _NC, _CH)
    dst3 = edge_index[1].reshape(_NT, _NC, _CH)
    ea = edge_attr.reshape(_E * _DE // _D, _D)

    ve1 = l1_We @ l1_att_edge
    ve2 = l2_We @ l2_att_edge
    m1 = jnp.kron(jnp.eye(8, dtype=f32), ve1.reshape(_DE, 1))
    m2 = jnp.kron(jnp.eye(8, dtype=f32), ve2.reshape(_DE, 1))
    ve = jnp.concatenate([m1, m2], axis=1)

    att1 = jnp.zeros((_D, _D), f32).at[:, 0].set(l1_att_src).at[:, 1].set(l1_att_dst)
    att2 = jnp.zeros((_D, _D), f32).at[:, 0].set(l2_att_src).at[:, 1].set(l2_att_dst)

    h1, asd1, aep, s1, mx2 = _a1_call(x, ea, l1_W, att1, ve)
    ae1 = lax.bitcast_convert_type(
        aep[:, 0:8].reshape(_NT, _NC, _CH), jnp.int32)
    ae2 = lax.bitcast_convert_type(
        aep[:, 8:16].reshape(_NT, _NC, _CH), jnp.int32)
    sdae1 = jnp.stack([src3, dst3, ae1], axis=2)
    sdae2 = jnp.stack([src3, dst3, ae2], axis=2)

    acc1, den1 = _sc_call(h1, sdae1, asd1[:, 0], asd1[:, 1], s1)
    denr1 = den1.T.reshape(_NB, _BN, 2)

    h2, asd2, s2 = _a2_call(acc1, denr1, l1_b.reshape(1, _D), l2_W, att2, mx2)

    acc2, den2 = _sc_call(h2, sdae2, asd2[:, 0], asd2[:, 1], s2)
    denr2 = den2.T.reshape(_NB, _BN, 2)

    return _e2_call(acc2, denr2, l2_b.reshape(1, _D),
                    ln_gamma.reshape(1, _D), ln_beta.reshape(1, _D))
